# baseline (speedup 1.0000x reference)
import numpy as np

import jax
import jax.numpy as jnp
from jax import lax
from jax.experimental import pallas as pl
from jax.experimental.pallas import tpu as pltpu
from jax.experimental.pallas import tpu_sc as plsc

N = 10000
E = 160000
G = 64
D_IN = 1056
H1, H2, H3, D_OUT = 256, 128, 64, 3

NC, NS = 2, 16
CH = 125
EC_AGG = E // NS
NCH_AGG = EC_AGG // CH
EC_DEG = E // (NS * NC)
NCH_DEG = EC_DEG // CH
ROWS_FULL = 640
ROWS_LAST = N - (NS - 1) * ROWS_FULL
ZROWS = 80

MB = 512
GRID = (N + MB - 1) // MB

_mesh = plsc.VectorSubcoreMesh(
    core_axis_name="c", subcore_axis_name="s", num_cores=NC, num_subcores=NS)
_sc_params = pltpu.CompilerParams(
    needs_layout_passes=False, use_tc_tiling_on_sc=False)


def _deg_body(c_hbm, w_hbm, out_hbm, c_v, w_v, zb, deg_sh):
    cid = lax.axis_index("c")
    sid = lax.axis_index("s")
    wid = cid * NS + sid
    pltpu.sync_copy(c_hbm.at[wid], c_v)
    pltpu.sync_copy(w_hbm.at[wid], w_v)
    for i in range(ROWS_FULL // 16):
        zb[pl.ds(i * 16, 16)] = jnp.zeros((16,), jnp.float32)
    row0 = sid * ROWS_FULL

    @pl.when(sid < NS - 1)
    def _():
        pltpu.sync_copy(zb, deg_sh.at[pl.ds(row0, ROWS_FULL)])

    @pl.when(sid == NS - 1)
    def _():
        pltpu.sync_copy(zb.at[pl.ds(0, ROWS_LAST)], deg_sh.at[pl.ds(row0, ROWS_LAST)])

    plsc.subcore_barrier()

    @pl.loop(0, NCH_DEG)
    def _scatter(j):
        pltpu.sync_copy(w_v.at[j], deg_sh.at[c_v.at[j]], add=True)

    plsc.subcore_barrier()
    base = cid * N + row0

    @pl.when(sid < NS - 1)
    def _():
        pltpu.sync_copy(deg_sh.at[pl.ds(row0, ROWS_FULL)], zb)
        pltpu.sync_copy(zb, out_hbm.at[pl.ds(base, ROWS_FULL)])

    @pl.when(sid == NS - 1)
    def _():
        pltpu.sync_copy(deg_sh.at[pl.ds(row0, ROWS_LAST)], zb.at[pl.ds(0, ROWS_LAST)])
        pltpu.sync_copy(zb.at[pl.ds(0, ROWS_LAST)], out_hbm.at[pl.ds(base, ROWS_LAST)])


_deg_call = pl.kernel(
    _deg_body,
    out_type=jax.ShapeDtypeStruct((NC * N,), jnp.float32),
    mesh=_mesh,
    compiler_params=_sc_params,
    scratch_types=[
        pltpu.VMEM((NCH_DEG, CH), jnp.int32),
        pltpu.VMEM((NCH_DEG, CH), jnp.float32),
        pltpu.VMEM((ROWS_FULL,), jnp.float32),
        pltpu.MemorySpace.VMEM_SHARED((N,), jnp.float32),
    ],
)


def _make_agg(dh, full_idx=False):
    mask_hi = np.int32(-65536)

    def scale_rows(buf16, fbuf, w_v, j):
        iot2 = lax.iota(jnp.int32, 16) * 2
        cols = [(iot2 + 32 * k, iot2 + (32 * k + 1)) for k in range(dh // 32)]

        @pl.loop(0, CH, unroll=5)
        def _row(e):
            fv = jnp.full((16,), j * CH + e, jnp.int32)
            ev = jnp.full((16,), e, jnp.int32)
            wv = plsc.load_gather(w_v, [fv])
            for k in range(dh // 32):
                v32 = buf16[e, pl.ds(32 * k, 32)]
                vi = plsc.bitcast(v32, jnp.int32)
                lo = plsc.bitcast(lax.shift_left(vi, 16), jnp.float32)
                hi = plsc.bitcast(lax.bitwise_and(vi, mask_hi), jnp.float32)
                plsc.store_scatter(fbuf, [ev, cols[k][0]], lo * wv)
                plsc.store_scatter(fbuf, [ev, cols[k][1]], hi * wv)

    def body(xs_a, xs_b, rc_hbm, w_hbm, out_a, out_b,
             rc_v, w_v, buf0, buf1, fbuf, agg_sh,
             isem0, isem1, gsem0, gsem1):
        cid = lax.axis_index("c")
        sid = lax.axis_index("s")
        pltpu.sync_copy(w_hbm.at[sid], w_v)

        @pl.loop(0, CH)
        def _z(i):
            for k in range(dh // 16):
                fbuf[i, pl.ds(k * 16, 16)] = jnp.zeros((16,), jnp.float32)

        zblk = fbuf.at[pl.ds(0, ZROWS)]
        row0 = sid * ROWS_FULL

        @pl.when(sid < NS - 1)
        def _():
            @pl.loop(0, ROWS_FULL // ZROWS)
            def _c(i):
                pltpu.sync_copy(zblk, agg_sh.at[pl.ds(row0 + i * ZROWS, ZROWS)])

        @pl.when(sid == NS - 1)
        def _():
            @pl.loop(0, ROWS_LAST // ZROWS)
            def _c(i):
                pltpu.sync_copy(zblk, agg_sh.at[pl.ds(row0 + i * ZROWS, ZROWS)])

        plsc.subcore_barrier()
        half = NCH_AGG // 2

        def run_ring(xs_hbm):
            pltpu.sync_copy(rc_hbm.at[sid, 0], rc_v.at[0])
            pltpu.make_async_copy(xs_hbm.at[rc_v.at[0, 0]], buf0,
                                  gsem0).start()
            pltpu.make_async_copy(rc_hbm.at[sid, 1], rc_v.at[1], isem1).start()

            @pl.loop(0, half)
            def _outer(t):
                j0 = 2 * t
                j1 = j0 + 1
                pltpu.make_async_copy(xs_hbm.at[rc_v.at[0, 0]], buf0,
                                      gsem0).wait()
                pltpu.make_async_copy(rc_hbm.at[sid, 0], rc_v.at[1],
                                      isem1).wait()
                pltpu.make_async_copy(xs_hbm.at[rc_v.at[1, 0]], buf1,
                                      gsem1).start()
                scale_rows(buf0, fbuf, w_v, j0)
                pltpu.sync_copy(fbuf, agg_sh.at[rc_v.at[0, 1]], add=True)

                @pl.when(t < half - 1)
                def _():
                    pltpu.make_async_copy(rc_hbm.at[sid, j0 + 2], rc_v.at[0],
                                          isem0).start()

                pltpu.make_async_copy(xs_hbm.at[rc_v.at[1, 0]], buf1,
                                      gsem1).wait()

                @pl.when(t < half - 1)
                def _():
                    pltpu.make_async_copy(rc_hbm.at[sid, 0], rc_v.at[0],
                                          isem0).wait()
                    pltpu.make_async_copy(xs_hbm.at[rc_v.at[0, 0]], buf0,
                                          gsem0).start()

                scale_rows(buf1, fbuf, w_v, j1)
                pltpu.sync_copy(fbuf, agg_sh.at[rc_v.at[1, 1]], add=True)

                @pl.when(t < half - 1)
                def _():
                    pltpu.make_async_copy(rc_hbm.at[sid, j1 + 2], rc_v.at[1],
                                          isem1).start()

        def run_full(xs_hbm):
            pltpu.sync_copy(rc_hbm.at[sid], rc_v)
            pltpu.make_async_copy(xs_hbm.at[rc_v.at[0, 0]], buf0,
                                  gsem0).start()

            @pl.loop(0, half)
            def _outer(t):
                j0 = 2 * t
                j1 = j0 + 1
                pltpu.make_async_copy(xs_hbm.at[rc_v.at[j0, 0]], buf0,
                                      gsem0).wait()
                pltpu.make_async_copy(xs_hbm.at[rc_v.at[j1, 0]], buf1,
                                      gsem1).start()
                scale_rows(buf0, fbuf, w_v, j0)
                pltpu.sync_copy(fbuf, agg_sh.at[rc_v.at[j0, 1]], add=True)
                pltpu.make_async_copy(xs_hbm.at[rc_v.at[j1, 0]], buf1,
                                      gsem1).wait()

                @pl.when(t < half - 1)
                def _():
                    pltpu.make_async_copy(xs_hbm.at[rc_v.at[j1 + 1, 0]], buf0,
                                          gsem0).start()

                scale_rows(buf1, fbuf, w_v, j1)
                pltpu.sync_copy(fbuf, agg_sh.at[rc_v.at[j1, 1]], add=True)

        run = run_full if full_idx else run_ring

        @pl.when(cid == 0)
        def _():
            run(xs_a)

        @pl.when(cid == 1)
        def _():
            run(xs_b)

        plsc.subcore_barrier()
        zblk2 = fbuf.at[pl.ds(0, ZROWS)]

        def drain(out_hbm):
            @pl.when(sid < NS - 1)
            def _():
                @pl.loop(0, ROWS_FULL // ZROWS)
                def _d(i):
                    pltpu.sync_copy(agg_sh.at[pl.ds(row0 + i * ZROWS, ZROWS)], zblk2)
                    pltpu.sync_copy(zblk2, out_hbm.at[pl.ds(row0 + i * ZROWS, ZROWS)])

            @pl.when(sid == NS - 1)
            def _():
                @pl.loop(0, ROWS_LAST // ZROWS)
                def _d(i):
                    pltpu.sync_copy(agg_sh.at[pl.ds(row0 + i * ZROWS, ZROWS)], zblk2)
                    pltpu.sync_copy(zblk2, out_hbm.at[pl.ds(row0 + i * ZROWS, ZROWS)])

        @pl.when(cid == 0)
        def _():
            drain(out_a)

        @pl.when(cid == 1)
        def _():
            drain(out_b)

    rc_shape = (NCH_AGG, 2, CH) if full_idx else (2, 2, CH)
    return pl.kernel(
        body,
        out_type=[jax.ShapeDtypeStruct((N, dh), jnp.float32)] * 2,
        mesh=_mesh,
        compiler_params=_sc_params,
        scratch_types=[
            pltpu.VMEM(rc_shape, jnp.int32),
            pltpu.VMEM((EC_AGG,), jnp.float32),
            pltpu.VMEM((CH, dh), jnp.bfloat16),
            pltpu.VMEM((CH, dh), jnp.bfloat16),
            pltpu.VMEM((CH, dh), jnp.float32),
            pltpu.MemorySpace.VMEM_SHARED((N, dh), jnp.float32),
            pltpu.SemaphoreType.DMA,
            pltpu.SemaphoreType.DMA,
            pltpu.SemaphoreType.DMA,
            pltpu.SemaphoreType.DMA,
        ],
    )


_agg = {H1 // 2: _make_agg(H1 // 2, full_idx=False),
        H2 // 2: _make_agg(H2 // 2, full_idx=True),
        H3 // 2: _make_agg(H3 // 2, full_idx=True)}


def _dis_of(deg_ref):
    d = deg_ref[...]
    s = d[:, 0:1] + d[:, 1:2] + 1.0
    return jnp.where(s > 0, lax.rsqrt(s), 0.0)


def _tc1(x, W1, degt):
    dh = H1 // 2

    def body(x_ref, w_ref, deg_ref, oa_ref, ob_ref, ba_ref, bb_ref):
        dis = _dis_of(deg_ref)
        xw = jnp.dot(x_ref[...], w_ref[...], preferred_element_type=jnp.float32)
        xs = xw * dis
        a = xs[:, :dh]
        b = xs[:, dh:]
        oa_ref[...] = a
        ob_ref[...] = b
        ba_ref[...] = a.astype(jnp.bfloat16)
        bb_ref[...] = b.astype(jnp.bfloat16)

    return pl.pallas_call(
        body,
        grid=(GRID,),
        in_specs=[
            pl.BlockSpec((MB, D_IN), lambda i: (i, 0)),
            pl.BlockSpec((D_IN, H1), lambda i: (0, 0)),
            pl.BlockSpec((MB, 2), lambda i: (i, 0)),
        ],
        out_specs=[pl.BlockSpec((MB, dh), lambda i: (i, 0))] * 4,
        out_shape=[jax.ShapeDtypeStruct((N, dh), jnp.float32)] * 2
        + [jax.ShapeDtypeStruct((N, dh), jnp.bfloat16)] * 2,
    )(x, W1, degt)


def _tc_mid(agg_a, agg_b, xs_a, xs_b, degt, b, W, din, dout):
    dhi, dho = din // 2, dout // 2

    def body(aa, ab, xa, xb, deg_ref, b_ref, w_ref,
             oa_ref, ob_ref, ba_ref, bb_ref):
        dis = _dis_of(deg_ref)
        aggf = jnp.concatenate([aa[...], ab[...]], axis=1)
        xsf = jnp.concatenate([xa[...], xb[...]], axis=1)
        h = jax.nn.relu(dis * (aggf + xsf) + b_ref[...])
        xw = jnp.dot(h, w_ref[...], preferred_element_type=jnp.float32)
        xs2 = xw * dis
        a = xs2[:, :dho]
        b2 = xs2[:, dho:]
        oa_ref[...] = a
        ob_ref[...] = b2
        ba_ref[...] = a.astype(jnp.bfloat16)
        bb_ref[...] = b2.astype(jnp.bfloat16)

    return pl.pallas_call(
        body,
        grid=(GRID,),
        in_specs=[
            pl.BlockSpec((MB, dhi), lambda i: (i, 0)),
            pl.BlockSpec((MB, dhi), lambda i: (i, 0)),
            pl.BlockSpec((MB, dhi), lambda i: (i, 0)),
            pl.BlockSpec((MB, dhi), lambda i: (i, 0)),
            pl.BlockSpec((MB, 2), lambda i: (i, 0)),
            pl.BlockSpec((1, din), lambda i: (0, 0)),
            pl.BlockSpec((din, dout), lambda i: (0, 0)),
        ],
        out_specs=[pl.BlockSpec((MB, dho), lambda i: (i, 0))] * 4,
        out_shape=[jax.ShapeDtypeStruct((N, dho), jnp.float32)] * 2
        + [jax.ShapeDtypeStruct((N, dho), jnp.bfloat16)] * 2,
    )(agg_a, agg_b, xs_a, xs_b, degt, b, W)


def _tc_final(agg_a, agg_b, xs_a, xs_b, degt, b3, batch2, Wl, bl):
    dhi = H3 // 2

    def body(aa, ab, xa, xb, deg_ref, b_ref, bt_ref, wl_ref, bl_ref,
             out_ref, sums_ref, cnts_ref):
        i = pl.program_id(0)

        @pl.when(i == 0)
        def _():
            sums_ref[...] = jnp.zeros_like(sums_ref)
            cnts_ref[...] = jnp.zeros_like(cnts_ref)

        dis = _dis_of(deg_ref)
        aggf = jnp.concatenate([aa[...], ab[...]], axis=1)
        xsf = jnp.concatenate([xa[...], xb[...]], axis=1)
        h = dis * (aggf + xsf) + b_ref[...]
        y = jnp.dot(h, wl_ref[...], preferred_element_type=jnp.float32)

        validr = (lax.broadcasted_iota(jnp.int32, (MB, 1), 0) + i * MB) < N
        validc = (lax.broadcasted_iota(jnp.int32, (1, MB), 1) + i * MB) < N
        ym = jnp.where(validr, y, 0.0)
        oh = (lax.broadcasted_iota(jnp.int32, (G, MB), 0) == bt_ref[...]).astype(jnp.float32)
        ohm = jnp.where(validc, oh, 0.0)
        sums_ref[...] += jnp.dot(ohm, ym, preferred_element_type=jnp.float32)
        cnts_ref[...] += jnp.sum(ohm, axis=1, keepdims=True)

        @pl.when(i == GRID - 1)
        def _():
            out_ref[...] = (sums_ref[...] / jnp.maximum(cnts_ref[...], 1.0)
                            + bl_ref[...])

    out, _, _ = pl.pallas_call(
        body,
        grid=(GRID,),
        in_specs=[
            pl.BlockSpec((MB, dhi), lambda i: (i, 0)),
            pl.BlockSpec((MB, dhi), lambda i: (i, 0)),
            pl.BlockSpec((MB, dhi), lambda i: (i, 0)),
            pl.BlockSpec((MB, dhi), lambda i: (i, 0)),
            pl.BlockSpec((MB, 2), lambda i: (i, 0)),
            pl.BlockSpec((1, H3), lambda i: (0, 0)),
            pl.BlockSpec((1, MB), lambda i: (0, i)),
            pl.BlockSpec((H3, D_OUT), lambda i: (0, 0)),
            pl.BlockSpec((1, D_OUT), lambda i: (0, 0)),
        ],
        out_specs=[
            pl.BlockSpec((G, D_OUT), lambda i: (0, 0)),
            pl.BlockSpec((G, D_OUT), lambda i: (0, 0)),
            pl.BlockSpec((G, 1), lambda i: (0, 0)),
        ],
        out_shape=[
            jax.ShapeDtypeStruct((G, D_OUT), jnp.float32),
            jax.ShapeDtypeStruct((G, D_OUT), jnp.float32),
            jax.ShapeDtypeStruct((G, 1), jnp.float32),
        ],
    )(agg_a, agg_b, xs_a, xs_b, degt, b3, batch2, Wl, bl)
    return out


def kernel(x, edge_index, edge_weight, batch, W1, b1, W2, b2, W3, b3, Wl, bl):
    r = edge_index[0]
    c = edge_index[1]
    rc_agg = jnp.stack(
        [r.reshape(NS, NCH_AGG, CH), c.reshape(NS, NCH_AGG, CH)], axis=2)
    w_agg = edge_weight.reshape(NS, EC_AGG)
    c_deg = c.reshape(NS * NC, NCH_DEG, CH)
    w_deg = edge_weight.reshape(NS * NC, NCH_DEG, CH)

    degp = _deg_call(c_deg, w_deg)
    degt = degp.reshape(NC, N).T

    xs1a, xs1b, bf1a, bf1b = _tc1(x, W1, degt)
    agg1a, agg1b = _agg[H1 // 2](bf1a, bf1b, rc_agg, w_agg)
    xs2a, xs2b, bf2a, bf2b = _tc_mid(agg1a, agg1b, xs1a, xs1b, degt,
                                     b1.reshape(1, H1), W2, H1, H2)
    agg2a, agg2b = _agg[H2 // 2](bf2a, bf2b, rc_agg, w_agg)
    xs3a, xs3b, bf3a, bf3b = _tc_mid(agg2a, agg2b, xs2a, xs2b, degt,
                                     b2.reshape(1, H2), W3, H2, H3)
    agg3a, agg3b = _agg[H3 // 2](bf3a, bf3b, rc_agg, w_agg)
    out = _tc_final(agg3a, agg3b, xs3a, xs3b, degt,
                    b3.reshape(1, H3), batch.reshape(1, N), Wl,
                    bl.reshape(1, D_OUT))
    return out


# speedup vs baseline: 1.0054x; 1.0054x over previous
import numpy as np

import jax
import jax.numpy as jnp
from jax import lax
from jax.experimental import pallas as pl
from jax.experimental.pallas import tpu as pltpu
from jax.experimental.pallas import tpu_sc as plsc

N = 10000
E = 160000
G = 64
D_IN = 1056
H1, H2, H3, D_OUT = 256, 128, 64, 3

NC, NS = 2, 16
CH = 125
EC_AGG = E // NS
NCH_AGG = EC_AGG // CH
EC_DEG = E // (NS * NC)
NCH_DEG = EC_DEG // CH
ROWS_FULL = 640
ROWS_LAST = N - (NS - 1) * ROWS_FULL
ZROWS = 80

MB = 512
GRID = (N + MB - 1) // MB

def _qinv(d):
    p = np.empty((d,), np.int32)
    for k in range(d // 32):
        for i in range(16):
            p[32 * k + 2 * i] = 32 * k + i
            p[32 * k + 2 * i + 1] = 32 * k + 16 + i
    return p


_mesh = plsc.VectorSubcoreMesh(
    core_axis_name="c", subcore_axis_name="s", num_cores=NC, num_subcores=NS)
_sc_params = pltpu.CompilerParams(
    needs_layout_passes=False, use_tc_tiling_on_sc=False)


def _deg_body(c_hbm, w_hbm, out_hbm, c_v, w_v, zb, deg_sh):
    cid = lax.axis_index("c")
    sid = lax.axis_index("s")
    wid = cid * NS + sid
    pltpu.sync_copy(c_hbm.at[wid], c_v)
    pltpu.sync_copy(w_hbm.at[wid], w_v)
    for i in range(ROWS_FULL // 16):
        zb[pl.ds(i * 16, 16)] = jnp.zeros((16,), jnp.float32)
    row0 = sid * ROWS_FULL

    @pl.when(sid < NS - 1)
    def _():
        pltpu.sync_copy(zb, deg_sh.at[pl.ds(row0, ROWS_FULL)])

    @pl.when(sid == NS - 1)
    def _():
        pltpu.sync_copy(zb.at[pl.ds(0, ROWS_LAST)], deg_sh.at[pl.ds(row0, ROWS_LAST)])

    plsc.subcore_barrier()

    @pl.loop(0, NCH_DEG)
    def _scatter(j):
        pltpu.sync_copy(w_v.at[j], deg_sh.at[c_v.at[j]], add=True)

    plsc.subcore_barrier()
    base = cid * N + row0

    @pl.when(sid < NS - 1)
    def _():
        pltpu.sync_copy(deg_sh.at[pl.ds(row0, ROWS_FULL)], zb)
        pltpu.sync_copy(zb, out_hbm.at[pl.ds(base, ROWS_FULL)])

    @pl.when(sid == NS - 1)
    def _():
        pltpu.sync_copy(deg_sh.at[pl.ds(row0, ROWS_LAST)], zb.at[pl.ds(0, ROWS_LAST)])
        pltpu.sync_copy(zb.at[pl.ds(0, ROWS_LAST)], out_hbm.at[pl.ds(base, ROWS_LAST)])


_deg_call = pl.kernel(
    _deg_body,
    out_type=jax.ShapeDtypeStruct((NC * N,), jnp.float32),
    mesh=_mesh,
    compiler_params=_sc_params,
    scratch_types=[
        pltpu.VMEM((NCH_DEG, CH), jnp.int32),
        pltpu.VMEM((NCH_DEG, CH), jnp.float32),
        pltpu.VMEM((ROWS_FULL,), jnp.float32),
        pltpu.MemorySpace.VMEM_SHARED((N,), jnp.float32),
    ],
)


def _make_agg(dh, full_idx=False):
    mask_hi = np.int32(-65536)

    def scale_rows(buf16, fbuf, w_v, j):
        @pl.loop(0, CH, unroll=5)
        def _row(e):
            fv = jnp.full((16,), j * CH + e, jnp.int32)
            wv = plsc.load_gather(w_v, [fv])
            for k in range(dh // 32):
                v32 = buf16[e, pl.ds(32 * k, 32)]
                vi = plsc.bitcast(v32, jnp.int32)
                lo = plsc.bitcast(lax.shift_left(vi, 16), jnp.float32)
                hi = plsc.bitcast(lax.bitwise_and(vi, mask_hi), jnp.float32)
                fbuf[e, pl.ds(32 * k, 16)] = lo * wv
                fbuf[e, pl.ds(32 * k + 16, 16)] = hi * wv

    def body(xs_a, xs_b, rc_hbm, w_hbm, out_a, out_b,
             rc_v, w_v, buf0, buf1, fbuf, agg_sh,
             isem0, isem1, gsem0, gsem1):
        cid = lax.axis_index("c")
        sid = lax.axis_index("s")
        pltpu.sync_copy(w_hbm.at[sid], w_v)

        @pl.loop(0, CH)
        def _z(i):
            for k in range(dh // 16):
                fbuf[i, pl.ds(k * 16, 16)] = jnp.zeros((16,), jnp.float32)

        zblk = fbuf.at[pl.ds(0, ZROWS)]
        row0 = sid * ROWS_FULL

        @pl.when(sid < NS - 1)
        def _():
            @pl.loop(0, ROWS_FULL // ZROWS)
            def _c(i):
                pltpu.sync_copy(zblk, agg_sh.at[pl.ds(row0 + i * ZROWS, ZROWS)])

        @pl.when(sid == NS - 1)
        def _():
            @pl.loop(0, ROWS_LAST // ZROWS)
            def _c(i):
                pltpu.sync_copy(zblk, agg_sh.at[pl.ds(row0 + i * ZROWS, ZROWS)])

        plsc.subcore_barrier()
        half = NCH_AGG // 2

        def run_ring(xs_hbm):
            pltpu.sync_copy(rc_hbm.at[sid, 0], rc_v.at[0])
            pltpu.make_async_copy(xs_hbm.at[rc_v.at[0, 0]], buf0,
                                  gsem0).start()
            pltpu.make_async_copy(rc_hbm.at[sid, 1], rc_v.at[1], isem1).start()

            @pl.loop(0, half)
            def _outer(t):
                j0 = 2 * t
                j1 = j0 + 1
                pltpu.make_async_copy(xs_hbm.at[rc_v.at[0, 0]], buf0,
                                      gsem0).wait()
                pltpu.make_async_copy(rc_hbm.at[sid, 0], rc_v.at[1],
                                      isem1).wait()
                pltpu.make_async_copy(xs_hbm.at[rc_v.at[1, 0]], buf1,
                                      gsem1).start()
                scale_rows(buf0, fbuf, w_v, j0)
                pltpu.sync_copy(fbuf, agg_sh.at[rc_v.at[0, 1]], add=True)

                @pl.when(t < half - 1)
                def _():
                    pltpu.make_async_copy(rc_hbm.at[sid, j0 + 2], rc_v.at[0],
                                          isem0).start()

                pltpu.make_async_copy(xs_hbm.at[rc_v.at[1, 0]], buf1,
                                      gsem1).wait()

                @pl.when(t < half - 1)
                def _():
                    pltpu.make_async_copy(rc_hbm.at[sid, 0], rc_v.at[0],
                                          isem0).wait()
                    pltpu.make_async_copy(xs_hbm.at[rc_v.at[0, 0]], buf0,
                                          gsem0).start()

                scale_rows(buf1, fbuf, w_v, j1)
                pltpu.sync_copy(fbuf, agg_sh.at[rc_v.at[1, 1]], add=True)

                @pl.when(t < half - 1)
                def _():
                    pltpu.make_async_copy(rc_hbm.at[sid, j1 + 2], rc_v.at[1],
                                          isem1).start()

        def run_full(xs_hbm):
            pltpu.sync_copy(rc_hbm.at[sid], rc_v)
            pltpu.make_async_copy(xs_hbm.at[rc_v.at[0, 0]], buf0,
                                  gsem0).start()

            @pl.loop(0, half)
            def _outer(t):
                j0 = 2 * t
                j1 = j0 + 1
                pltpu.make_async_copy(xs_hbm.at[rc_v.at[j0, 0]], buf0,
                                      gsem0).wait()
                pltpu.make_async_copy(xs_hbm.at[rc_v.at[j1, 0]], buf1,
                                      gsem1).start()
                scale_rows(buf0, fbuf, w_v, j0)
                pltpu.sync_copy(fbuf, agg_sh.at[rc_v.at[j0, 1]], add=True)
                pltpu.make_async_copy(xs_hbm.at[rc_v.at[j1, 0]], buf1,
                                      gsem1).wait()

                @pl.when(t < half - 1)
                def _():
                    pltpu.make_async_copy(xs_hbm.at[rc_v.at[j1 + 1, 0]], buf0,
                                          gsem0).start()

                scale_rows(buf1, fbuf, w_v, j1)
                pltpu.sync_copy(fbuf, agg_sh.at[rc_v.at[j1, 1]], add=True)

        run = run_full if full_idx else run_ring

        @pl.when(cid == 0)
        def _():
            run(xs_a)

        @pl.when(cid == 1)
        def _():
            run(xs_b)

        plsc.subcore_barrier()
        zblk2 = fbuf.at[pl.ds(0, ZROWS)]

        def drain(out_hbm):
            @pl.when(sid < NS - 1)
            def _():
                @pl.loop(0, ROWS_FULL // ZROWS)
                def _d(i):
                    pltpu.sync_copy(agg_sh.at[pl.ds(row0 + i * ZROWS, ZROWS)], zblk2)
                    pltpu.sync_copy(zblk2, out_hbm.at[pl.ds(row0 + i * ZROWS, ZROWS)])

            @pl.when(sid == NS - 1)
            def _():
                @pl.loop(0, ROWS_LAST // ZROWS)
                def _d(i):
                    pltpu.sync_copy(agg_sh.at[pl.ds(row0 + i * ZROWS, ZROWS)], zblk2)
                    pltpu.sync_copy(zblk2, out_hbm.at[pl.ds(row0 + i * ZROWS, ZROWS)])

        @pl.when(cid == 0)
        def _():
            drain(out_a)

        @pl.when(cid == 1)
        def _():
            drain(out_b)

    rc_shape = (NCH_AGG, 2, CH) if full_idx else (2, 2, CH)
    return pl.kernel(
        body,
        out_type=[jax.ShapeDtypeStruct((N, dh), jnp.float32)] * 2,
        mesh=_mesh,
        compiler_params=_sc_params,
        scratch_types=[
            pltpu.VMEM(rc_shape, jnp.int32),
            pltpu.VMEM((EC_AGG,), jnp.float32),
            pltpu.VMEM((CH, dh), jnp.bfloat16),
            pltpu.VMEM((CH, dh), jnp.bfloat16),
            pltpu.VMEM((CH, dh), jnp.float32),
            pltpu.MemorySpace.VMEM_SHARED((N, dh), jnp.float32),
            pltpu.SemaphoreType.DMA,
            pltpu.SemaphoreType.DMA,
            pltpu.SemaphoreType.DMA,
            pltpu.SemaphoreType.DMA,
        ],
    )


_agg = {H1 // 2: _make_agg(H1 // 2, full_idx=False),
        H2 // 2: _make_agg(H2 // 2, full_idx=True),
        H3 // 2: _make_agg(H3 // 2, full_idx=True)}


def _dis_of(deg_ref):
    d = deg_ref[...]
    s = d[:, 0:1] + d[:, 1:2] + 1.0
    return jnp.where(s > 0, lax.rsqrt(s), 0.0)


def _shuf(a, q_ref):
    idx = jnp.broadcast_to(q_ref[0:1, :], a.shape)
    return jnp.take_along_axis(a, idx, axis=1).astype(jnp.bfloat16)


def _tc1(x, W1, degt, qv):
    dh = H1 // 2

    def body(x_ref, w_ref, deg_ref, q_ref, oa_ref, ob_ref, ba_ref, bb_ref):
        dis = _dis_of(deg_ref)
        xw = jnp.dot(x_ref[...], w_ref[...], preferred_element_type=jnp.float32)
        xs = xw * dis
        a = xs[:, :dh]
        b = xs[:, dh:]
        oa_ref[...] = a
        ob_ref[...] = b
        ba_ref[...] = _shuf(a, q_ref)
        bb_ref[...] = _shuf(b, q_ref)

    return pl.pallas_call(
        body,
        grid=(GRID,),
        in_specs=[
            pl.BlockSpec((MB, D_IN), lambda i: (i, 0)),
            pl.BlockSpec((D_IN, H1), lambda i: (0, 0)),
            pl.BlockSpec((MB, 2), lambda i: (i, 0)),
            pl.BlockSpec((1, dh), lambda i: (0, 0)),
        ],
        out_specs=[pl.BlockSpec((MB, dh), lambda i: (i, 0))] * 4,
        out_shape=[jax.ShapeDtypeStruct((N, dh), jnp.float32)] * 2
        + [jax.ShapeDtypeStruct((N, dh), jnp.bfloat16)] * 2,
    )(x, W1, degt, qv)


def _tc_mid(agg_a, agg_b, xs_a, xs_b, degt, b, W, din, dout, qv):
    dhi, dho = din // 2, dout // 2

    def body(aa, ab, xa, xb, deg_ref, b_ref, w_ref, q_ref,
             oa_ref, ob_ref, ba_ref, bb_ref):
        dis = _dis_of(deg_ref)
        aggf = jnp.concatenate([aa[...], ab[...]], axis=1)
        xsf = jnp.concatenate([xa[...], xb[...]], axis=1)
        h = jax.nn.relu(dis * (aggf + xsf) + b_ref[...])
        xw = jnp.dot(h, w_ref[...], preferred_element_type=jnp.float32)
        xs2 = xw * dis
        a = xs2[:, :dho]
        b2 = xs2[:, dho:]
        oa_ref[...] = a
        ob_ref[...] = b2
        ba_ref[...] = _shuf(a, q_ref)
        bb_ref[...] = _shuf(b2, q_ref)

    return pl.pallas_call(
        body,
        grid=(GRID,),
        in_specs=[
            pl.BlockSpec((MB, dhi), lambda i: (i, 0)),
            pl.BlockSpec((MB, dhi), lambda i: (i, 0)),
            pl.BlockSpec((MB, dhi), lambda i: (i, 0)),
            pl.BlockSpec((MB, dhi), lambda i: (i, 0)),
            pl.BlockSpec((MB, 2), lambda i: (i, 0)),
            pl.BlockSpec((1, din), lambda i: (0, 0)),
            pl.BlockSpec((din, dout), lambda i: (0, 0)),
            pl.BlockSpec((1, dho), lambda i: (0, 0)),
        ],
        out_specs=[pl.BlockSpec((MB, dho), lambda i: (i, 0))] * 4,
        out_shape=[jax.ShapeDtypeStruct((N, dho), jnp.float32)] * 2
        + [jax.ShapeDtypeStruct((N, dho), jnp.bfloat16)] * 2,
    )(agg_a, agg_b, xs_a, xs_b, degt, b, W, qv)


def _tc_final(agg_a, agg_b, xs_a, xs_b, degt, b3, batch2, Wl, bl):
    dhi = H3 // 2

    def body(aa, ab, xa, xb, deg_ref, b_ref, bt_ref, wl_ref, bl_ref,
             out_ref, sums_ref, cnts_ref):
        i = pl.program_id(0)

        @pl.when(i == 0)
        def _():
            sums_ref[...] = jnp.zeros_like(sums_ref)
            cnts_ref[...] = jnp.zeros_like(cnts_ref)

        dis = _dis_of(deg_ref)
        aggf = jnp.concatenate([aa[...], ab[...]], axis=1)
        xsf = jnp.concatenate([xa[...], xb[...]], axis=1)
        h = dis * (aggf + xsf) + b_ref[...]
        y = jnp.dot(h, wl_ref[...], preferred_element_type=jnp.float32)

        validr = (lax.broadcasted_iota(jnp.int32, (MB, 1), 0) + i * MB) < N
        validc = (lax.broadcasted_iota(jnp.int32, (1, MB), 1) + i * MB) < N
        ym = jnp.where(validr, y, 0.0)
        oh = (lax.broadcasted_iota(jnp.int32, (G, MB), 0) == bt_ref[...]).astype(jnp.float32)
        ohm = jnp.where(validc, oh, 0.0)
        sums_ref[...] += jnp.dot(ohm, ym, preferred_element_type=jnp.float32)
        cnts_ref[...] += jnp.sum(ohm, axis=1, keepdims=True)

        @pl.when(i == GRID - 1)
        def _():
            out_ref[...] = (sums_ref[...] / jnp.maximum(cnts_ref[...], 1.0)
                            + bl_ref[...])

    out, _, _ = pl.pallas_call(
        body,
        grid=(GRID,),
        in_specs=[
            pl.BlockSpec((MB, dhi), lambda i: (i, 0)),
            pl.BlockSpec((MB, dhi), lambda i: (i, 0)),
            pl.BlockSpec((MB, dhi), lambda i: (i, 0)),
            pl.BlockSpec((MB, dhi), lambda i: (i, 0)),
            pl.BlockSpec((MB, 2), lambda i: (i, 0)),
            pl.BlockSpec((1, H3), lambda i: (0, 0)),
            pl.BlockSpec((1, MB), lambda i: (0, i)),
            pl.BlockSpec((H3, D_OUT), lambda i: (0, 0)),
            pl.BlockSpec((1, D_OUT), lambda i: (0, 0)),
        ],
        out_specs=[
            pl.BlockSpec((G, D_OUT), lambda i: (0, 0)),
            pl.BlockSpec((G, D_OUT), lambda i: (0, 0)),
            pl.BlockSpec((G, 1), lambda i: (0, 0)),
        ],
        out_shape=[
            jax.ShapeDtypeStruct((G, D_OUT), jnp.float32),
            jax.ShapeDtypeStruct((G, D_OUT), jnp.float32),
            jax.ShapeDtypeStruct((G, 1), jnp.float32),
        ],
    )(agg_a, agg_b, xs_a, xs_b, degt, b3, batch2, Wl, bl)
    return out


def kernel(x, edge_index, edge_weight, batch, W1, b1, W2, b2, W3, b3, Wl, bl):
    r = edge_index[0]
    c = edge_index[1]
    rc_agg = jnp.stack(
        [r.reshape(NS, NCH_AGG, CH), c.reshape(NS, NCH_AGG, CH)], axis=2)
    w_agg = edge_weight.reshape(NS, EC_AGG)
    c_deg = c.reshape(NS * NC, NCH_DEG, CH)
    w_deg = edge_weight.reshape(NS * NC, NCH_DEG, CH)

    degp = _deg_call(c_deg, w_deg)
    degt = degp.reshape(NC, N).T
    q1 = jnp.asarray(_qinv(H1 // 2)).reshape(1, H1 // 2)
    q2 = jnp.asarray(_qinv(H2 // 2)).reshape(1, H2 // 2)
    q3 = jnp.asarray(_qinv(H3 // 2)).reshape(1, H3 // 2)

    xs1a, xs1b, bf1a, bf1b = _tc1(x, W1, degt, q1)
    agg1a, agg1b = _agg[H1 // 2](bf1a, bf1b, rc_agg, w_agg)
    xs2a, xs2b, bf2a, bf2b = _tc_mid(agg1a, agg1b, xs1a, xs1b, degt,
                                     b1.reshape(1, H1), W2, H1, H2, q2)
    agg2a, agg2b = _agg[H2 // 2](bf2a, bf2b, rc_agg, w_agg)
    xs3a, xs3b, bf3a, bf3b = _tc_mid(agg2a, agg2b, xs2a, xs2b, degt,
                                     b2.reshape(1, H2), W3, H2, H3, q3)
    agg3a, agg3b = _agg[H3 // 2](bf3a, bf3b, rc_agg, w_agg)
    out = _tc_final(agg3a, agg3b, xs3a, xs3b, degt,
                    b3.reshape(1, H3), batch.reshape(1, N), Wl,
                    bl.reshape(1, D_OUT))
    return out


# speedup vs baseline: 1.2702x; 1.2634x over previous
import numpy as np

import jax
import jax.numpy as jnp
from jax import lax
from jax.experimental import pallas as pl
from jax.experimental.pallas import tpu as pltpu
from jax.experimental.pallas import tpu_sc as plsc

N = 10000
E = 160000
G = 64
D_IN = 1056
H1, H2, H3, D_OUT = 256, 128, 64, 3

NC, NS = 2, 16
CH = 125
EC_AGG = E // NS
NCH_AGG = EC_AGG // CH
EC_DEG = E // (NS * NC)
NCH_DEG = EC_DEG // CH
ROWS_FULL = 640
ROWS_LAST = N - (NS - 1) * ROWS_FULL
ZROWS = 80

MB = 512
GRID = (N + MB - 1) // MB

def _qinv(d):
    p = np.empty((d,), np.int32)
    for k in range(d // 32):
        for i in range(16):
            p[32 * k + 2 * i] = 32 * k + i
            p[32 * k + 2 * i + 1] = 32 * k + 16 + i
    return p


_mesh = plsc.VectorSubcoreMesh(
    core_axis_name="c", subcore_axis_name="s", num_cores=NC, num_subcores=NS)
_sc_params = pltpu.CompilerParams(
    needs_layout_passes=False, use_tc_tiling_on_sc=False)


def _deg_body(c_hbm, w_hbm, out_hbm, c_v, w_v, zb, deg_sh):
    cid = lax.axis_index("c")
    sid = lax.axis_index("s")
    wid = cid * NS + sid
    pltpu.sync_copy(c_hbm.at[wid], c_v)
    pltpu.sync_copy(w_hbm.at[wid], w_v)
    for i in range(ROWS_FULL // 16):
        zb[pl.ds(i * 16, 16)] = jnp.zeros((16,), jnp.float32)
    row0 = sid * ROWS_FULL

    @pl.when(sid < NS - 1)
    def _():
        pltpu.sync_copy(zb, deg_sh.at[pl.ds(row0, ROWS_FULL)])

    @pl.when(sid == NS - 1)
    def _():
        pltpu.sync_copy(zb.at[pl.ds(0, ROWS_LAST)], deg_sh.at[pl.ds(row0, ROWS_LAST)])

    plsc.subcore_barrier()

    @pl.loop(0, NCH_DEG)
    def _scatter(j):
        pltpu.sync_copy(w_v.at[j], deg_sh.at[c_v.at[j]], add=True)

    plsc.subcore_barrier()
    base = cid * N + row0

    @pl.when(sid < NS - 1)
    def _():
        pltpu.sync_copy(deg_sh.at[pl.ds(row0, ROWS_FULL)], zb)
        pltpu.sync_copy(zb, out_hbm.at[pl.ds(base, ROWS_FULL)])

    @pl.when(sid == NS - 1)
    def _():
        pltpu.sync_copy(deg_sh.at[pl.ds(row0, ROWS_LAST)], zb.at[pl.ds(0, ROWS_LAST)])
        pltpu.sync_copy(zb.at[pl.ds(0, ROWS_LAST)], out_hbm.at[pl.ds(base, ROWS_LAST)])


_deg_call = pl.kernel(
    _deg_body,
    out_type=jax.ShapeDtypeStruct((NC * N,), jnp.float32),
    mesh=_mesh,
    compiler_params=_sc_params,
    scratch_types=[
        pltpu.VMEM((NCH_DEG, CH), jnp.int32),
        pltpu.VMEM((NCH_DEG, CH), jnp.float32),
        pltpu.VMEM((ROWS_FULL,), jnp.float32),
        pltpu.MemorySpace.VMEM_SHARED((N,), jnp.float32),
    ],
)


def _make_agg(dh, full_idx=False):
    mask_hi = np.int32(-65536)

    def scale_rows(buf16, fbuf, w_v, j):
        @pl.loop(0, CH, unroll=5)
        def _row(e):
            fv = jnp.full((16,), j * CH + e, jnp.int32)
            wv = plsc.load_gather(w_v, [fv])
            vis = [plsc.bitcast(buf16[e, pl.ds(32 * k, 32)], jnp.int32)
                   for k in range(dh // 32)]
            los = [plsc.bitcast(lax.shift_left(vi, 16), jnp.float32) * wv
                   for vi in vis]
            his = [plsc.bitcast(lax.bitwise_and(vi, mask_hi), jnp.float32) * wv
                   for vi in vis]
            for k in range(dh // 32):
                fbuf[e, pl.ds(32 * k, 16)] = los[k]
                fbuf[e, pl.ds(32 * k + 16, 16)] = his[k]

    def body(xs_a, xs_b, rc_hbm, w_hbm, out_a, out_b,
             rc_v, w_v, buf0, buf1, fbuf, agg_sh,
             isem0, isem1, gsem0, gsem1):
        cid = lax.axis_index("c")
        sid = lax.axis_index("s")
        pltpu.sync_copy(w_hbm.at[sid], w_v)

        @pl.loop(0, CH)
        def _z(i):
            for k in range(dh // 16):
                fbuf[i, pl.ds(k * 16, 16)] = jnp.zeros((16,), jnp.float32)

        zblk = fbuf.at[pl.ds(0, ZROWS)]
        row0 = sid * ROWS_FULL

        @pl.when(sid < NS - 1)
        def _():
            @pl.loop(0, ROWS_FULL // ZROWS)
            def _c(i):
                pltpu.sync_copy(zblk, agg_sh.at[pl.ds(row0 + i * ZROWS, ZROWS)])

        @pl.when(sid == NS - 1)
        def _():
            @pl.loop(0, ROWS_LAST // ZROWS)
            def _c(i):
                pltpu.sync_copy(zblk, agg_sh.at[pl.ds(row0 + i * ZROWS, ZROWS)])

        plsc.subcore_barrier()
        half = NCH_AGG // 2

        def run_ring(xs_hbm):
            pltpu.sync_copy(rc_hbm.at[sid, 0], rc_v.at[0])
            pltpu.make_async_copy(xs_hbm.at[rc_v.at[0, 0]], buf0,
                                  gsem0).start()
            pltpu.make_async_copy(rc_hbm.at[sid, 1], rc_v.at[1], isem1).start()

            @pl.loop(0, half)
            def _outer(t):
                j0 = 2 * t
                j1 = j0 + 1
                pltpu.make_async_copy(xs_hbm.at[rc_v.at[0, 0]], buf0,
                                      gsem0).wait()
                pltpu.make_async_copy(rc_hbm.at[sid, 0], rc_v.at[1],
                                      isem1).wait()
                pltpu.make_async_copy(xs_hbm.at[rc_v.at[1, 0]], buf1,
                                      gsem1).start()
                scale_rows(buf0, fbuf, w_v, j0)
                pltpu.sync_copy(fbuf, agg_sh.at[rc_v.at[0, 1]], add=True)

                @pl.when(t < half - 1)
                def _():
                    pltpu.make_async_copy(rc_hbm.at[sid, j0 + 2], rc_v.at[0],
                                          isem0).start()

                pltpu.make_async_copy(xs_hbm.at[rc_v.at[1, 0]], buf1,
                                      gsem1).wait()

                @pl.when(t < half - 1)
                def _():
                    pltpu.make_async_copy(rc_hbm.at[sid, 0], rc_v.at[0],
                                          isem0).wait()
                    pltpu.make_async_copy(xs_hbm.at[rc_v.at[0, 0]], buf0,
                                          gsem0).start()

                scale_rows(buf1, fbuf, w_v, j1)
                pltpu.sync_copy(fbuf, agg_sh.at[rc_v.at[1, 1]], add=True)

                @pl.when(t < half - 1)
                def _():
                    pltpu.make_async_copy(rc_hbm.at[sid, j1 + 2], rc_v.at[1],
                                          isem1).start()

        def run_full(xs_hbm):
            pltpu.sync_copy(rc_hbm.at[sid], rc_v)
            pltpu.make_async_copy(xs_hbm.at[rc_v.at[0, 0]], buf0,
                                  gsem0).start()

            @pl.loop(0, half)
            def _outer(t):
                j0 = 2 * t
                j1 = j0 + 1
                pltpu.make_async_copy(xs_hbm.at[rc_v.at[j0, 0]], buf0,
                                      gsem0).wait()
                pltpu.make_async_copy(xs_hbm.at[rc_v.at[j1, 0]], buf1,
                                      gsem1).start()
                scale_rows(buf0, fbuf, w_v, j0)
                pltpu.sync_copy(fbuf, agg_sh.at[rc_v.at[j0, 1]], add=True)
                pltpu.make_async_copy(xs_hbm.at[rc_v.at[j1, 0]], buf1,
                                      gsem1).wait()

                @pl.when(t < half - 1)
                def _():
                    pltpu.make_async_copy(xs_hbm.at[rc_v.at[j1 + 1, 0]], buf0,
                                          gsem0).start()

                scale_rows(buf1, fbuf, w_v, j1)
                pltpu.sync_copy(fbuf, agg_sh.at[rc_v.at[j1, 1]], add=True)

        run = run_full if full_idx else run_ring

        @pl.when(cid == 0)
        def _():
            run(xs_a)

        @pl.when(cid == 1)
        def _():
            run(xs_b)

        plsc.subcore_barrier()
        zblk2 = fbuf.at[pl.ds(0, ZROWS)]

        def drain(out_hbm):
            @pl.when(sid < NS - 1)
            def _():
                @pl.loop(0, ROWS_FULL // ZROWS)
                def _d(i):
                    pltpu.sync_copy(agg_sh.at[pl.ds(row0 + i * ZROWS, ZROWS)], zblk2)
                    pltpu.sync_copy(zblk2, out_hbm.at[pl.ds(row0 + i * ZROWS, ZROWS)])

            @pl.when(sid == NS - 1)
            def _():
                @pl.loop(0, ROWS_LAST // ZROWS)
                def _d(i):
                    pltpu.sync_copy(agg_sh.at[pl.ds(row0 + i * ZROWS, ZROWS)], zblk2)
                    pltpu.sync_copy(zblk2, out_hbm.at[pl.ds(row0 + i * ZROWS, ZROWS)])

        @pl.when(cid == 0)
        def _():
            drain(out_a)

        @pl.when(cid == 1)
        def _():
            drain(out_b)

    rc_shape = (NCH_AGG, 2, CH) if full_idx else (2, 2, CH)
    return pl.kernel(
        body,
        out_type=[jax.ShapeDtypeStruct((N, dh), jnp.float32)] * 2,
        mesh=_mesh,
        compiler_params=_sc_params,
        scratch_types=[
            pltpu.VMEM(rc_shape, jnp.int32),
            pltpu.VMEM((EC_AGG,), jnp.float32),
            pltpu.VMEM((CH, dh), jnp.bfloat16),
            pltpu.VMEM((CH, dh), jnp.bfloat16),
            pltpu.VMEM((CH, dh), jnp.float32),
            pltpu.MemorySpace.VMEM_SHARED((N, dh), jnp.float32),
            pltpu.SemaphoreType.DMA,
            pltpu.SemaphoreType.DMA,
            pltpu.SemaphoreType.DMA,
            pltpu.SemaphoreType.DMA,
        ],
    )


_agg = {H1 // 2: _make_agg(H1 // 2, full_idx=False),
        H2 // 2: _make_agg(H2 // 2, full_idx=True),
        H3 // 2: _make_agg(H3 // 2, full_idx=True)}


def _dis_of(deg_ref):
    d = deg_ref[...]
    s = d[:, 0:1] + d[:, 1:2] + 1.0
    return jnp.where(s > 0, lax.rsqrt(s), 0.0)


def _shuf(a, q_ref):
    idx = jnp.broadcast_to(q_ref[0:1, :], a.shape)
    return jnp.take_along_axis(a, idx, axis=1).astype(jnp.bfloat16)


def _tc1(x, W1, degt, qv):
    dh = H1 // 2

    def body(x_ref, w_ref, deg_ref, q_ref, oa_ref, ob_ref, ba_ref, bb_ref):
        dis = _dis_of(deg_ref)
        xw = jnp.dot(x_ref[...], w_ref[...], preferred_element_type=jnp.float32)
        xs = xw * dis
        a = xs[:, :dh]
        b = xs[:, dh:]
        oa_ref[...] = a
        ob_ref[...] = b
        ba_ref[...] = _shuf(a, q_ref)
        bb_ref[...] = _shuf(b, q_ref)

    return pl.pallas_call(
        body,
        grid=(GRID,),
        in_specs=[
            pl.BlockSpec((MB, D_IN), lambda i: (i, 0)),
            pl.BlockSpec((D_IN, H1), lambda i: (0, 0)),
            pl.BlockSpec((MB, 2), lambda i: (i, 0)),
            pl.BlockSpec((1, dh), lambda i: (0, 0)),
        ],
        out_specs=[pl.BlockSpec((MB, dh), lambda i: (i, 0))] * 4,
        out_shape=[jax.ShapeDtypeStruct((N, dh), jnp.float32)] * 2
        + [jax.ShapeDtypeStruct((N, dh), jnp.bfloat16)] * 2,
    )(x, W1, degt, qv)


def _tc_mid(agg_a, agg_b, xs_a, xs_b, degt, b, W, din, dout, qv):
    dhi, dho = din // 2, dout // 2

    def body(aa, ab, xa, xb, deg_ref, b_ref, w_ref, q_ref,
             oa_ref, ob_ref, ba_ref, bb_ref):
        dis = _dis_of(deg_ref)
        aggf = jnp.concatenate([aa[...], ab[...]], axis=1)
        xsf = jnp.concatenate([xa[...], xb[...]], axis=1)
        h = jax.nn.relu(dis * (aggf + xsf) + b_ref[...])
        xw = jnp.dot(h, w_ref[...], preferred_element_type=jnp.float32)
        xs2 = xw * dis
        a = xs2[:, :dho]
        b2 = xs2[:, dho:]
        oa_ref[...] = a
        ob_ref[...] = b2
        ba_ref[...] = _shuf(a, q_ref)
        bb_ref[...] = _shuf(b2, q_ref)

    return pl.pallas_call(
        body,
        grid=(GRID,),
        in_specs=[
            pl.BlockSpec((MB, dhi), lambda i: (i, 0)),
            pl.BlockSpec((MB, dhi), lambda i: (i, 0)),
            pl.BlockSpec((MB, dhi), lambda i: (i, 0)),
            pl.BlockSpec((MB, dhi), lambda i: (i, 0)),
            pl.BlockSpec((MB, 2), lambda i: (i, 0)),
            pl.BlockSpec((1, din), lambda i: (0, 0)),
            pl.BlockSpec((din, dout), lambda i: (0, 0)),
            pl.BlockSpec((1, dho), lambda i: (0, 0)),
        ],
        out_specs=[pl.BlockSpec((MB, dho), lambda i: (i, 0))] * 4,
        out_shape=[jax.ShapeDtypeStruct((N, dho), jnp.float32)] * 2
        + [jax.ShapeDtypeStruct((N, dho), jnp.bfloat16)] * 2,
    )(agg_a, agg_b, xs_a, xs_b, degt, b, W, qv)


def _tc_final(agg_a, agg_b, xs_a, xs_b, degt, b3, batch2, Wl, bl):
    dhi = H3 // 2

    def body(aa, ab, xa, xb, deg_ref, b_ref, bt_ref, wl_ref, bl_ref,
             out_ref, sums_ref, cnts_ref):
        i = pl.program_id(0)

        @pl.when(i == 0)
        def _():
            sums_ref[...] = jnp.zeros_like(sums_ref)
            cnts_ref[...] = jnp.zeros_like(cnts_ref)

        dis = _dis_of(deg_ref)
        aggf = jnp.concatenate([aa[...], ab[...]], axis=1)
        xsf = jnp.concatenate([xa[...], xb[...]], axis=1)
        h = dis * (aggf + xsf) + b_ref[...]
        y = jnp.dot(h, wl_ref[...], preferred_element_type=jnp.float32)

        validr = (lax.broadcasted_iota(jnp.int32, (MB, 1), 0) + i * MB) < N
        validc = (lax.broadcasted_iota(jnp.int32, (1, MB), 1) + i * MB) < N
        ym = jnp.where(validr, y, 0.0)
        oh = (lax.broadcasted_iota(jnp.int32, (G, MB), 0) == bt_ref[...]).astype(jnp.float32)
        ohm = jnp.where(validc, oh, 0.0)
        sums_ref[...] += jnp.dot(ohm, ym, preferred_element_type=jnp.float32)
        cnts_ref[...] += jnp.sum(ohm, axis=1, keepdims=True)

        @pl.when(i == GRID - 1)
        def _():
            out_ref[...] = (sums_ref[...] / jnp.maximum(cnts_ref[...], 1.0)
                            + bl_ref[...])

    out, _, _ = pl.pallas_call(
        body,
        grid=(GRID,),
        in_specs=[
            pl.BlockSpec((MB, dhi), lambda i: (i, 0)),
            pl.BlockSpec((MB, dhi), lambda i: (i, 0)),
            pl.BlockSpec((MB, dhi), lambda i: (i, 0)),
            pl.BlockSpec((MB, dhi), lambda i: (i, 0)),
            pl.BlockSpec((MB, 2), lambda i: (i, 0)),
            pl.BlockSpec((1, H3), lambda i: (0, 0)),
            pl.BlockSpec((1, MB), lambda i: (0, i)),
            pl.BlockSpec((H3, D_OUT), lambda i: (0, 0)),
            pl.BlockSpec((1, D_OUT), lambda i: (0, 0)),
        ],
        out_specs=[
            pl.BlockSpec((G, D_OUT), lambda i: (0, 0)),
            pl.BlockSpec((G, D_OUT), lambda i: (0, 0)),
            pl.BlockSpec((G, 1), lambda i: (0, 0)),
        ],
        out_shape=[
            jax.ShapeDtypeStruct((G, D_OUT), jnp.float32),
            jax.ShapeDtypeStruct((G, D_OUT), jnp.float32),
            jax.ShapeDtypeStruct((G, 1), jnp.float32),
        ],
    )(agg_a, agg_b, xs_a, xs_b, degt, b3, batch2, Wl, bl)
    return out


def kernel(x, edge_index, edge_weight, batch, W1, b1, W2, b2, W3, b3, Wl, bl):
    r = edge_index[0]
    c = edge_index[1]
    rc_agg = jnp.stack(
        [r.reshape(NS, NCH_AGG, CH), c.reshape(NS, NCH_AGG, CH)], axis=2)
    w_agg = edge_weight.reshape(NS, EC_AGG)
    c_deg = c.reshape(NS * NC, NCH_DEG, CH)
    w_deg = edge_weight.reshape(NS * NC, NCH_DEG, CH)

    degp = _deg_call(c_deg, w_deg)
    degt = degp.reshape(NC, N).T
    q1 = jnp.asarray(_qinv(H1 // 2)).reshape(1, H1 // 2)
    q2 = jnp.asarray(_qinv(H2 // 2)).reshape(1, H2 // 2)
    q3 = jnp.asarray(_qinv(H3 // 2)).reshape(1, H3 // 2)

    xs1a, xs1b, bf1a, bf1b = _tc1(x, W1, degt, q1)
    agg1a, agg1b = _agg[H1 // 2](bf1a, bf1b, rc_agg, w_agg)
    xs2a, xs2b, bf2a, bf2b = _tc_mid(agg1a, agg1b, xs1a, xs1b, degt,
                                     b1.reshape(1, H1), W2, H1, H2, q2)
    agg2a, agg2b = _agg[H2 // 2](bf2a, bf2b, rc_agg, w_agg)
    xs3a, xs3b, bf3a, bf3b = _tc_mid(agg2a, agg2b, xs2a, xs2b, degt,
                                     b2.reshape(1, H2), W3, H2, H3, q3)
    agg3a, agg3b = _agg[H3 // 2](bf3a, bf3b, rc_agg, w_agg)
    out = _tc_final(agg3a, agg3b, xs3a, xs3b, degt,
                    b3.reshape(1, H3), batch.reshape(1, N), Wl,
                    bl.reshape(1, D_OUT))
    return out
